# Initial kernel scaffold; baseline (speedup 1.0000x reference)
import jax
import jax.numpy as jnp
from jax.experimental import pallas as pl


def kernel(x, edge_index, W0, b0, W1, b1, W2, b2, W3, b3, W4, b4):
    raise NotImplementedError("write your pallas kernel here")


# speedup vs baseline: 37.7932x; 37.7932x over previous
import functools

import jax
import jax.numpy as jnp
from jax import lax
from jax.experimental import pallas as pl
from jax.experimental.pallas import tpu as pltpu
from jax.experimental.pallas import tpu_sc as plsc

N = 10000
E = 320000
DIN = 128
DH = 64
NPAD = 10240
NP2 = NPAD // 2
NW = 32
NB = 80
BB = 128
EPAD = NW * NB * BB
RPT = NPAD // 16
RP = 512
NRB = NP2 // RP

_MESH = plsc.VectorSubcoreMesh(
    core_axis_name="c", subcore_axis_name="s", num_cores=2, num_subcores=16
)

_SC_PARAMS = pltpu.CompilerParams(use_tc_tiling_on_sc=False)


def _deg_body(dst_hbm, ones_hbm, zeros_hbm, out_hbm, didx, ones_v, acc, ssem):
    c = lax.axis_index("c")
    s = lax.axis_index("s")
    wid = c * 16 + s
    pltpu.sync_copy(dst_hbm.at[wid], didx)
    pltpu.sync_copy(ones_hbm, ones_v)
    r0 = s * RPT
    pltpu.sync_copy(zeros_hbm, acc.at[pl.ds(r0, RPT)])
    plsc.subcore_barrier()

    def fire(b, carry):
        pltpu.async_copy(ones_v, acc.at[didx.at[b]], ssem, add=True)
        return carry

    lax.fori_loop(0, NB, fire, 0)

    def drain(b, carry):
        pltpu.make_async_copy(ones_v, acc.at[didx.at[0]], ssem).wait()
        return carry

    lax.fori_loop(0, NB, drain, 0)
    plsc.subcore_barrier()
    pltpu.sync_copy(acc.at[pl.ds(r0, RPT)], out_hbm.at[c, pl.ds(r0, RPT)])


_sc_deg = functools.partial(
    pl.kernel,
    out_type=jax.ShapeDtypeStruct((2, NPAD, DH), jnp.float32),
    mesh=_MESH,
    compiler_params=_SC_PARAMS,
    scratch_types=[
        pltpu.VMEM((NB, BB), jnp.int32),
        pltpu.VMEM((BB, DH), jnp.float32),
        pltpu.VMEM_SHARED((NPAD, DH), jnp.float32),
        pltpu.SemaphoreType.DMA,
    ],
)(_deg_body)


def _prop_body(g_hbm, src_hbm, dst_hbm, out_hbm, sidx, didx, rows0, rows1,
               rows2, rows3, acc, gs0, gs1, gs2, gs3, ss0, ss1, ss2, ss3):
    c = lax.axis_index("c")
    s = lax.axis_index("s")
    wid = c * 16 + s
    pltpu.sync_copy(src_hbm.at[wid], sidx)
    pltpu.sync_copy(dst_hbm.at[wid], didx)
    r0 = s * RPT
    pltpu.sync_copy(g_hbm.at[pl.ds(r0, RPT)], acc.at[pl.ds(r0, RPT)])
    plsc.subcore_barrier()

    rows = (rows0, rows1, rows2, rows3)
    gsem = (gs0, gs1, gs2, gs3)
    ssem = (ss0, ss1, ss2, ss3)

    def fire_gather(b, k):
        pltpu.async_copy(g_hbm.at[sidx.at[b]], rows[k], gsem[k])

    def wait_gather(b, k):
        pltpu.make_async_copy(g_hbm.at[sidx.at[b]], rows[k], gsem[k]).wait()

    def fire_scatter(b, k):
        pltpu.async_copy(rows[k], acc.at[didx.at[b]], ssem[k], add=True)

    def wait_scatter(b, k):
        pltpu.make_async_copy(rows[k], acc.at[didx.at[b]], ssem[k]).wait()

    def step(b, k, do_ss_wait, do_g_fire):
        k2 = (k + 2) % 4
        if do_ss_wait:
            wait_scatter(b - 2, k2)
        if do_g_fire:
            fire_gather(b + 2, k2)
        wait_gather(b, k)
        fire_scatter(b, k)

    fire_gather(0, 0)
    fire_gather(1, 1)
    step(0, 0, False, True)
    step(1, 1, False, True)

    def quad(it, carry):
        b = 2 + 4 * it
        step(b + 0, 2, True, True)
        step(b + 1, 3, True, True)
        step(b + 2, 0, True, True)
        step(b + 3, 1, True, True)
        return carry

    lax.fori_loop(0, (NB - 4) // 4, quad, 0)
    step(NB - 2, (NB - 2) % 4, True, False)
    step(NB - 1, (NB - 1) % 4, True, False)
    wait_scatter(NB - 2, (NB - 2) % 4)
    wait_scatter(NB - 1, (NB - 1) % 4)
    plsc.subcore_barrier()
    pltpu.sync_copy(acc.at[pl.ds(r0, RPT)], out_hbm.at[c, pl.ds(r0, RPT)])


_sc_prop = functools.partial(
    pl.kernel,
    out_type=jax.ShapeDtypeStruct((2, NPAD, DH), jnp.float32),
    mesh=_MESH,
    compiler_params=_SC_PARAMS,
    scratch_types=[
        pltpu.VMEM((NB, BB), jnp.int32),
        pltpu.VMEM((NB, BB), jnp.int32),
        pltpu.VMEM((BB, DH), jnp.float32),
        pltpu.VMEM((BB, DH), jnp.float32),
        pltpu.VMEM((BB, DH), jnp.float32),
        pltpu.VMEM((BB, DH), jnp.float32),
        pltpu.VMEM_SHARED((NPAD, DH), jnp.float32),
        pltpu.SemaphoreType.DMA,
        pltpu.SemaphoreType.DMA,
        pltpu.SemaphoreType.DMA,
        pltpu.SemaphoreType.DMA,
        pltpu.SemaphoreType.DMA,
        pltpu.SemaphoreType.DMA,
        pltpu.SemaphoreType.DMA,
        pltpu.SemaphoreType.DMA,
    ],
)(_prop_body)


def _tc_dis_body(d_ref, o_ref):
    o_ref[...] = lax.rsqrt(1.0 + d_ref[0] + d_ref[1])


def _tc_first_body(x_ref, w_ref, dis_ref, g_ref):
    g_ref[...] = dis_ref[...] * jnp.dot(
        x_ref[...], w_ref[...], preferred_element_type=jnp.float32
    )


def _tc_comb_body(a_ref, g_ref, dis_ref, b_ref, w_ref, o_ref):
    dis = dis_ref[...]
    t = jnp.maximum(dis * (a_ref[0] + a_ref[1] - g_ref[...]) + b_ref[...], 0.0)
    o_ref[...] = dis * jnp.dot(t, w_ref[...], preferred_element_type=jnp.float32)


def _tc_last_body(a_ref, g_ref, dis_ref, b_ref, o_ref):
    o_ref[...] = jnp.maximum(
        dis_ref[...] * (a_ref[0] + a_ref[1] - g_ref[...]) + b_ref[...], 0.0
    )


def _row_spec(cols=128):
    return pl.BlockSpec((RP, cols), lambda i: (i, 0))


def _pair_spec():
    return pl.BlockSpec((2, RP, 128), lambda i: (0, i, 0))


def _full_spec(r, c):
    return pl.BlockSpec((r, c), lambda i: (0, 0))


_PAIR_OUT = jax.ShapeDtypeStruct((NP2, 128), jnp.float32)

_tc_dis = pl.pallas_call(
    _tc_dis_body,
    grid=(NRB,),
    in_specs=[_pair_spec()],
    out_specs=_row_spec(),
    out_shape=_PAIR_OUT,
)

_tc_first = pl.pallas_call(
    _tc_first_body,
    grid=(NRB,),
    in_specs=[_row_spec(256), _full_spec(256, 128), _row_spec()],
    out_specs=_row_spec(),
    out_shape=_PAIR_OUT,
)

_tc_comb = pl.pallas_call(
    _tc_comb_body,
    grid=(NRB,),
    in_specs=[
        _pair_spec(), _row_spec(), _row_spec(),
        _full_spec(1, 128), _full_spec(128, 128),
    ],
    out_specs=_row_spec(),
    out_shape=_PAIR_OUT,
)

_tc_last = pl.pallas_call(
    _tc_last_body,
    grid=(NRB,),
    in_specs=[_pair_spec(), _row_spec(), _row_spec(), _full_spec(1, 128)],
    out_specs=_row_spec(),
    out_shape=_PAIR_OUT,
)


def _blockdiag(W):
    di, do = W.shape
    Wb = jnp.zeros((2 * di, 128), W.dtype)
    return Wb.at[:di, :do].set(W).at[di:, do:].set(W)


def kernel(x, edge_index, W0, b0, W1, b1, W2, b2, W3, b3, W4, b4):
    src = edge_index[0].astype(jnp.int32)
    dst = edge_index[1].astype(jnp.int32)
    n_pad_rows = NPAD - N
    pad_ids = N + (jnp.arange(EPAD - E, dtype=jnp.int32) % n_pad_rows)
    src3 = jnp.concatenate([src, pad_ids]).reshape(NW, NB, BB)
    dst3 = jnp.concatenate([dst, pad_ids]).reshape(NW, NB, BB)
    x_pair = jnp.pad(x, ((0, n_pad_rows), (0, 0))).reshape(NP2, 2 * DIN)
    ones = jnp.ones((BB, DH), jnp.float32)
    zeros = jnp.zeros((RPT, DH), jnp.float32)

    degp = _sc_deg(dst3, ones, zeros)
    dis = _tc_dis(degp.reshape(2, NP2, 128))

    g = _tc_first(x_pair, _blockdiag(W0), dis)
    Ws = [W1, W2, W3, W4]
    bs = [b0, b1, b2, b3, b4]
    for i in range(4):
        acc = _sc_prop(g.reshape(NPAD, DH), src3, dst3)
        g = _tc_comb(acc.reshape(2, NP2, 128), g, dis,
                     jnp.concatenate([bs[i], bs[i]]).reshape(1, 128),
                     _blockdiag(Ws[i]))
    acc = _sc_prop(g.reshape(NPAD, DH), src3, dst3)
    out = _tc_last(acc.reshape(2, NP2, 128), g, dis,
                   jnp.concatenate([bs[4], bs[4]]).reshape(1, 128))
    return out.reshape(NPAD, DH)[:N]


# speedup vs baseline: 38.8860x; 1.0289x over previous
import functools

import jax
import jax.numpy as jnp
from jax import lax
from jax.experimental import pallas as pl
from jax.experimental.pallas import tpu as pltpu
from jax.experimental.pallas import tpu_sc as plsc

N = 10000
E = 320000
DIN = 128
DH = 64
NPAD = 10240
NP2 = NPAD // 2
NW = 32
NB = 80
BB = 128
EPAD = NW * NB * BB
RPT = NPAD // 16
RP = 512
NRB = NP2 // RP

_MESH = plsc.VectorSubcoreMesh(
    core_axis_name="c", subcore_axis_name="s", num_cores=2, num_subcores=16
)

_SC_PARAMS = pltpu.CompilerParams(use_tc_tiling_on_sc=False)


def _deg_body(dst_hbm, ones_hbm, zeros_hbm, out_hbm, didx, ones_v, acc, ssem):
    c = lax.axis_index("c")
    s = lax.axis_index("s")
    wid = c * 16 + s
    pltpu.sync_copy(dst_hbm.at[wid], didx)
    pltpu.sync_copy(ones_hbm, ones_v)
    r0 = s * RPT
    pltpu.sync_copy(zeros_hbm, acc.at[pl.ds(r0, RPT)])
    plsc.subcore_barrier()

    def fire(b, carry):
        pltpu.async_copy(ones_v, acc.at[didx.at[b]], ssem, add=True)
        return carry

    lax.fori_loop(0, NB, fire, 0)

    def drain(b, carry):
        pltpu.make_async_copy(ones_v, acc.at[didx.at[0]], ssem).wait()
        return carry

    lax.fori_loop(0, NB, drain, 0)
    plsc.subcore_barrier()
    pltpu.sync_copy(acc.at[pl.ds(r0, RPT)], out_hbm.at[c, pl.ds(r0, RPT)])


_sc_deg = functools.partial(
    pl.kernel,
    out_type=jax.ShapeDtypeStruct((2, NPAD, DH), jnp.float32),
    mesh=_MESH,
    compiler_params=_SC_PARAMS,
    scratch_types=[
        pltpu.VMEM((NB, BB), jnp.int32),
        pltpu.VMEM((BB, DH), jnp.float32),
        pltpu.VMEM_SHARED((NPAD, DH), jnp.float32),
        pltpu.SemaphoreType.DMA,
    ],
)(_deg_body)


def _prop_body(g_hbm, src_hbm, dst_hbm, out_hbm, sidx, didx, rows_v,
               acc, gs0, gs1, gs2, gs3, gs4, gs5, gs6, gs7,
               ss0, ss1, ss2, ss3, ss4, ss5, ss6, ss7):
    c = lax.axis_index("c")
    s = lax.axis_index("s")
    wid = c * 16 + s
    pltpu.sync_copy(src_hbm.at[wid], sidx)
    pltpu.sync_copy(dst_hbm.at[wid], didx)
    r0 = s * RPT
    pltpu.sync_copy(g_hbm.at[pl.ds(r0, RPT)], acc.at[pl.ds(r0, RPT)])
    plsc.subcore_barrier()

    NS = 8
    GA = 4
    gsem = (gs0, gs1, gs2, gs3, gs4, gs5, gs6, gs7)
    ssem = (ss0, ss1, ss2, ss3, ss4, ss5, ss6, ss7)

    def fire_gather(b, k):
        pltpu.async_copy(g_hbm.at[sidx.at[b]], rows_v.at[k], gsem[k])

    def wait_gather(b, k):
        pltpu.make_async_copy(g_hbm.at[sidx.at[b]], rows_v.at[k], gsem[k]).wait()

    def fire_scatter(b, k):
        pltpu.async_copy(rows_v.at[k], acc.at[didx.at[b]], ssem[k], add=True)

    def wait_scatter(b, k):
        pltpu.make_async_copy(rows_v.at[k], acc.at[didx.at[b]], ssem[k]).wait()

    def step(b, k, do_ss_wait, do_g_fire):
        k2 = (k + GA) % NS
        if do_ss_wait:
            wait_scatter(b - GA, k2)
        if do_g_fire:
            fire_gather(b + GA, k2)
        wait_gather(b, k)
        fire_scatter(b, k)

    for b in range(GA):
        fire_gather(b, b)
    for b in range(GA):
        step(b, b, False, True)

    def octet(it, carry):
        b = GA + NS * it
        for j in range(NS):
            step(b + j, (GA + j) % NS, True, True)
        return carry

    lax.fori_loop(0, (NB - 2 * GA) // NS, octet, 0)
    for b in range(NB - GA, NB):
        step(b, b % NS, True, False)
    for b in range(NB - GA, NB):
        wait_scatter(b, b % NS)
    plsc.subcore_barrier()
    pltpu.sync_copy(acc.at[pl.ds(r0, RPT)], out_hbm.at[c, pl.ds(r0, RPT)])


_sc_prop = functools.partial(
    pl.kernel,
    out_type=jax.ShapeDtypeStruct((2, NPAD, DH), jnp.float32),
    mesh=_MESH,
    compiler_params=_SC_PARAMS,
    scratch_types=[
        pltpu.VMEM((NB, BB), jnp.int32),
        pltpu.VMEM((NB, BB), jnp.int32),
        pltpu.VMEM((8, BB, DH), jnp.float32),
        pltpu.VMEM_SHARED((NPAD, DH), jnp.float32),
    ] + [pltpu.SemaphoreType.DMA] * 16,
)(_prop_body)


def _tc_dis_body(d_ref, o_ref):
    o_ref[...] = lax.rsqrt(1.0 + d_ref[0] + d_ref[1])


def _tc_first_body(x_ref, w_ref, dis_ref, g_ref):
    g_ref[...] = dis_ref[...] * jnp.dot(
        x_ref[...], w_ref[...], preferred_element_type=jnp.float32
    )


def _tc_comb_body(a_ref, g_ref, dis_ref, b_ref, w_ref, o_ref):
    dis = dis_ref[...]
    t = jnp.maximum(dis * (a_ref[0] + a_ref[1] - g_ref[...]) + b_ref[...], 0.0)
    o_ref[...] = dis * jnp.dot(t, w_ref[...], preferred_element_type=jnp.float32)


def _tc_last_body(a_ref, g_ref, dis_ref, b_ref, o_ref):
    o_ref[...] = jnp.maximum(
        dis_ref[...] * (a_ref[0] + a_ref[1] - g_ref[...]) + b_ref[...], 0.0
    )


def _row_spec(cols=128):
    return pl.BlockSpec((RP, cols), lambda i: (i, 0))


def _pair_spec():
    return pl.BlockSpec((2, RP, 128), lambda i: (0, i, 0))


def _full_spec(r, c):
    return pl.BlockSpec((r, c), lambda i: (0, 0))


_PAIR_OUT = jax.ShapeDtypeStruct((NP2, 128), jnp.float32)

_tc_dis = pl.pallas_call(
    _tc_dis_body,
    grid=(NRB,),
    in_specs=[_pair_spec()],
    out_specs=_row_spec(),
    out_shape=_PAIR_OUT,
)

_tc_first = pl.pallas_call(
    _tc_first_body,
    grid=(NRB,),
    in_specs=[_row_spec(256), _full_spec(256, 128), _row_spec()],
    out_specs=_row_spec(),
    out_shape=_PAIR_OUT,
)

_tc_comb = pl.pallas_call(
    _tc_comb_body,
    grid=(NRB,),
    in_specs=[
        _pair_spec(), _row_spec(), _row_spec(),
        _full_spec(1, 128), _full_spec(128, 128),
    ],
    out_specs=_row_spec(),
    out_shape=_PAIR_OUT,
)

_tc_last = pl.pallas_call(
    _tc_last_body,
    grid=(NRB,),
    in_specs=[_pair_spec(), _row_spec(), _row_spec(), _full_spec(1, 128)],
    out_specs=_row_spec(),
    out_shape=_PAIR_OUT,
)


def _blockdiag(W):
    di, do = W.shape
    Wb = jnp.zeros((2 * di, 128), W.dtype)
    return Wb.at[:di, :do].set(W).at[di:, do:].set(W)


def kernel(x, edge_index, W0, b0, W1, b1, W2, b2, W3, b3, W4, b4):
    src = edge_index[0].astype(jnp.int32)
    dst = edge_index[1].astype(jnp.int32)
    n_pad_rows = NPAD - N
    pad_ids = N + (jnp.arange(EPAD - E, dtype=jnp.int32) % n_pad_rows)
    src3 = jnp.concatenate([src, pad_ids]).reshape(NW, NB, BB)
    dst3 = jnp.concatenate([dst, pad_ids]).reshape(NW, NB, BB)
    x_pair = jnp.pad(x, ((0, n_pad_rows), (0, 0))).reshape(NP2, 2 * DIN)
    ones = jnp.ones((BB, DH), jnp.float32)
    zeros = jnp.zeros((RPT, DH), jnp.float32)

    degp = _sc_deg(dst3, ones, zeros)
    dis = _tc_dis(degp.reshape(2, NP2, 128))

    g = _tc_first(x_pair, _blockdiag(W0), dis)
    Ws = [W1, W2, W3, W4]
    bs = [b0, b1, b2, b3, b4]
    for i in range(4):
        acc = _sc_prop(g.reshape(NPAD, DH), src3, dst3)
        g = _tc_comb(acc.reshape(2, NP2, 128), g, dis,
                     jnp.concatenate([bs[i], bs[i]]).reshape(1, 128),
                     _blockdiag(Ws[i]))
    acc = _sc_prop(g.reshape(NPAD, DH), src3, dst3)
    out = _tc_last(acc.reshape(2, NP2, 128), g, dis,
                   jnp.concatenate([bs[4], bs[4]]).reshape(1, 128))
    return out.reshape(NPAD, DH)[:N]


# speedup vs baseline: 39.1269x; 1.0062x over previous
import functools

import jax
import jax.numpy as jnp
from jax import lax
from jax.experimental import pallas as pl
from jax.experimental.pallas import tpu as pltpu
from jax.experimental.pallas import tpu_sc as plsc

N = 10000
E = 320000
DIN = 128
DH = 64
NPAD = 10240
NP2 = NPAD // 2
NW = 32
NB = 80
BB = 128
EPAD = NW * NB * BB
RPT = NPAD // 16
RP = 512
NRB = NP2 // RP

_MESH = plsc.VectorSubcoreMesh(
    core_axis_name="c", subcore_axis_name="s", num_cores=2, num_subcores=16
)

_SC_PARAMS = pltpu.CompilerParams(use_tc_tiling_on_sc=False)


def _deg_body(dst_hbm, ones_hbm, zeros_hbm, out_hbm, didx, ones_v, acc, ssem):
    c = lax.axis_index("c")
    s = lax.axis_index("s")
    wid = c * 16 + s
    pltpu.sync_copy(dst_hbm.at[wid], didx)
    pltpu.sync_copy(ones_hbm, ones_v)
    r0 = s * RPT
    pltpu.sync_copy(zeros_hbm, acc.at[pl.ds(r0, RPT)])
    plsc.subcore_barrier()

    def fire(b, carry):
        pltpu.async_copy(ones_v, acc.at[didx.at[b]], ssem, add=True)
        return carry

    lax.fori_loop(0, NB, fire, 0)

    def drain(b, carry):
        pltpu.make_async_copy(ones_v, acc.at[didx.at[0]], ssem).wait()
        return carry

    lax.fori_loop(0, NB, drain, 0)
    plsc.subcore_barrier()
    pltpu.sync_copy(acc.at[pl.ds(r0, RPT)], out_hbm.at[c, pl.ds(r0, RPT)])


_sc_deg = functools.partial(
    pl.kernel,
    out_type=jax.ShapeDtypeStruct((2, NPAD, DH), jnp.float32),
    mesh=_MESH,
    compiler_params=_SC_PARAMS,
    scratch_types=[
        pltpu.VMEM((NB, BB), jnp.int32),
        pltpu.VMEM((BB, DH), jnp.float32),
        pltpu.VMEM_SHARED((NPAD, DH), jnp.float32),
        pltpu.SemaphoreType.DMA,
    ],
)(_deg_body)


def _prop_body(g_hbm, src_hbm, dst_hbm, out_hbm, sidx, didx, rows_v,
               acc, gs0, gs1, gs2, gs3, gs4, gs5, gs6, gs7,
               ss0, ss1, ss2, ss3, ss4, ss5, ss6, ss7):
    c = lax.axis_index("c")
    s = lax.axis_index("s")
    wid = c * 16 + s
    NS = 8
    GA = 4
    gsem = (gs0, gs1, gs2, gs3, gs4, gs5, gs6, gs7)
    ssem = (ss0, ss1, ss2, ss3, ss4, ss5, ss6, ss7)
    pltpu.sync_copy(src_hbm.at[wid], sidx)
    for b in range(GA):
        pltpu.async_copy(g_hbm.at[sidx.at[b]], rows_v.at[b], gsem[b])
    pltpu.sync_copy(dst_hbm.at[wid], didx)
    r0 = s * RPT
    pltpu.sync_copy(g_hbm.at[pl.ds(r0, RPT)], acc.at[pl.ds(r0, RPT)])
    plsc.subcore_barrier()

    def fire_gather(b, k):
        pltpu.async_copy(g_hbm.at[sidx.at[b]], rows_v.at[k], gsem[k])

    def wait_gather(b, k):
        pltpu.make_async_copy(g_hbm.at[sidx.at[b]], rows_v.at[k], gsem[k]).wait()

    def fire_scatter(b, k):
        pltpu.async_copy(rows_v.at[k], acc.at[didx.at[b]], ssem[k], add=True)

    def wait_scatter(b, k):
        pltpu.make_async_copy(rows_v.at[k], acc.at[didx.at[b]], ssem[k]).wait()

    def step(b, k, do_ss_wait, do_g_fire):
        k2 = (k + GA) % NS
        if do_ss_wait:
            wait_scatter(b - GA, k2)
        if do_g_fire:
            fire_gather(b + GA, k2)
        wait_gather(b, k)
        fire_scatter(b, k)

    for b in range(GA):
        step(b, b, False, True)

    def octet(it, carry):
        b = GA + NS * it
        for j in range(NS):
            step(b + j, (GA + j) % NS, True, True)
        return carry

    lax.fori_loop(0, (NB - 2 * GA) // NS, octet, 0)
    for b in range(NB - GA, NB):
        step(b, b % NS, True, False)
    for b in range(NB - GA, NB):
        wait_scatter(b, b % NS)
    plsc.subcore_barrier()
    pltpu.sync_copy(acc.at[pl.ds(r0, RPT)], out_hbm.at[c, pl.ds(r0, RPT)])


_sc_prop = functools.partial(
    pl.kernel,
    out_type=jax.ShapeDtypeStruct((2, NPAD, DH), jnp.float32),
    mesh=_MESH,
    compiler_params=_SC_PARAMS,
    scratch_types=[
        pltpu.VMEM((NB, BB), jnp.int32),
        pltpu.VMEM((NB, BB), jnp.int32),
        pltpu.VMEM((8, BB, DH), jnp.float32),
        pltpu.VMEM_SHARED((NPAD, DH), jnp.float32),
    ] + [pltpu.SemaphoreType.DMA] * 16,
)(_prop_body)


def _tc_dis_body(d_ref, o_ref):
    o_ref[...] = lax.rsqrt(1.0 + d_ref[0] + d_ref[1])


def _tc_first_body(x_ref, w_ref, dis_ref, g_ref):
    g_ref[...] = dis_ref[...] * jnp.dot(
        x_ref[...], w_ref[...], preferred_element_type=jnp.float32
    )


def _tc_comb_body(a_ref, g_ref, dis_ref, b_ref, w_ref, o_ref):
    dis = dis_ref[...]
    t = jnp.maximum(dis * (a_ref[0] + a_ref[1] - g_ref[...]) + b_ref[...], 0.0)
    o_ref[...] = dis * jnp.dot(t, w_ref[...], preferred_element_type=jnp.float32)


def _tc_last_body(a_ref, g_ref, dis_ref, b_ref, o_ref):
    o_ref[...] = jnp.maximum(
        dis_ref[...] * (a_ref[0] + a_ref[1] - g_ref[...]) + b_ref[...], 0.0
    )


def _row_spec(cols=128):
    return pl.BlockSpec((RP, cols), lambda i: (i, 0))


def _pair_spec():
    return pl.BlockSpec((2, RP, 128), lambda i: (0, i, 0))


def _full_spec(r, c):
    return pl.BlockSpec((r, c), lambda i: (0, 0))


_PAIR_OUT = jax.ShapeDtypeStruct((NP2, 128), jnp.float32)

_tc_dis = pl.pallas_call(
    _tc_dis_body,
    grid=(NRB,),
    in_specs=[_pair_spec()],
    out_specs=_row_spec(),
    out_shape=_PAIR_OUT,
)

_tc_first = pl.pallas_call(
    _tc_first_body,
    grid=(NRB,),
    in_specs=[_row_spec(256), _full_spec(256, 128), _row_spec()],
    out_specs=_row_spec(),
    out_shape=_PAIR_OUT,
)

_tc_comb = pl.pallas_call(
    _tc_comb_body,
    grid=(NRB,),
    in_specs=[
        _pair_spec(), _row_spec(), _row_spec(),
        _full_spec(1, 128), _full_spec(128, 128),
    ],
    out_specs=_row_spec(),
    out_shape=_PAIR_OUT,
)

_tc_last = pl.pallas_call(
    _tc_last_body,
    grid=(NRB,),
    in_specs=[_pair_spec(), _row_spec(), _row_spec(), _full_spec(1, 128)],
    out_specs=_row_spec(),
    out_shape=_PAIR_OUT,
)


def _blockdiag(W):
    di, do = W.shape
    Wb = jnp.zeros((2 * di, 128), W.dtype)
    return Wb.at[:di, :do].set(W).at[di:, do:].set(W)


def kernel(x, edge_index, W0, b0, W1, b1, W2, b2, W3, b3, W4, b4):
    src = edge_index[0].astype(jnp.int32)
    dst = edge_index[1].astype(jnp.int32)
    n_pad_rows = NPAD - N
    pad_ids = N + (jnp.arange(EPAD - E, dtype=jnp.int32) % n_pad_rows)
    src3 = jnp.concatenate([src, pad_ids]).reshape(NW, NB, BB)
    dst3 = jnp.concatenate([dst, pad_ids]).reshape(NW, NB, BB)
    x_pair = jnp.pad(x, ((0, n_pad_rows), (0, 0))).reshape(NP2, 2 * DIN)
    ones = jnp.ones((BB, DH), jnp.float32)
    zeros = jnp.zeros((RPT, DH), jnp.float32)

    degp = _sc_deg(dst3, ones, zeros)
    dis = _tc_dis(degp.reshape(2, NP2, 128))

    g = _tc_first(x_pair, _blockdiag(W0), dis)
    Ws = [W1, W2, W3, W4]
    bs = [b0, b1, b2, b3, b4]
    for i in range(4):
        acc = _sc_prop(g.reshape(NPAD, DH), src3, dst3)
        g = _tc_comb(acc.reshape(2, NP2, 128), g, dis,
                     jnp.concatenate([bs[i], bs[i]]).reshape(1, 128),
                     _blockdiag(Ws[i]))
    acc = _sc_prop(g.reshape(NPAD, DH), src3, dst3)
    out = _tc_last(acc.reshape(2, NP2, 128), g, dis,
                   jnp.concatenate([bs[4], bs[4]]).reshape(1, 128))
    return out.reshape(NPAD, DH)[:N]


# speedup vs baseline: 39.8112x; 1.0175x over previous
import functools

import jax
import jax.numpy as jnp
from jax import lax
from jax.experimental import pallas as pl
from jax.experimental.pallas import tpu as pltpu
from jax.experimental.pallas import tpu_sc as plsc

N = 10000
E = 320000
DIN = 128
DH = 64
NPAD = 10240
NP2 = NPAD // 2
NW = 32
NB = 80
BB = 128
EPAD = NW * NB * BB
RPT = NPAD // 16
RP = 512
NRB = NP2 // RP

_MESH = plsc.VectorSubcoreMesh(
    core_axis_name="c", subcore_axis_name="s", num_cores=2, num_subcores=16
)

_SC_PARAMS = pltpu.CompilerParams(use_tc_tiling_on_sc=False)


def _deg_body(edges_hbm, ones_hbm, zeros_hbm, out_hbm, didx, ones_v, acc, ssem):
    c = lax.axis_index("c")
    s = lax.axis_index("s")
    wid = c * 16 + s
    pltpu.sync_copy(edges_hbm.at[1, wid], didx)
    pltpu.sync_copy(ones_hbm, ones_v)
    r0 = s * RPT
    pltpu.sync_copy(zeros_hbm, acc.at[pl.ds(r0, RPT)])
    plsc.subcore_barrier()

    def fire(b, carry):
        pltpu.async_copy(ones_v, acc.at[didx.at[b]], ssem, add=True)
        return carry

    lax.fori_loop(0, NB, fire, 0)

    def drain(b, carry):
        pltpu.make_async_copy(ones_v, acc.at[didx.at[0]], ssem).wait()
        return carry

    lax.fori_loop(0, NB, drain, 0)
    plsc.subcore_barrier()
    pltpu.sync_copy(acc.at[pl.ds(r0, RPT)], out_hbm.at[c, pl.ds(r0, RPT)])


_sc_deg = functools.partial(
    pl.kernel,
    out_type=jax.ShapeDtypeStruct((2, NPAD, DH), jnp.float32),
    mesh=_MESH,
    compiler_params=_SC_PARAMS,
    scratch_types=[
        pltpu.VMEM((NB, BB), jnp.int32),
        pltpu.VMEM((BB, DH), jnp.float32),
        pltpu.VMEM_SHARED((NPAD, DH), jnp.float32),
        pltpu.SemaphoreType.DMA,
    ],
)(_deg_body)


def _prop_body(g_hbm, edges_hbm, out_hbm, sidx, didx, rows_v,
               acc, gs0, gs1, gs2, gs3, gs4, gs5, gs6, gs7,
               ss0, ss1, ss2, ss3, ss4, ss5, ss6, ss7):
    c = lax.axis_index("c")
    s = lax.axis_index("s")
    wid = c * 16 + s
    NS = 8
    GA = 4
    gsem = (gs0, gs1, gs2, gs3, gs4, gs5, gs6, gs7)
    ssem = (ss0, ss1, ss2, ss3, ss4, ss5, ss6, ss7)
    pltpu.sync_copy(edges_hbm.at[0, wid], sidx)
    for b in range(GA):
        pltpu.async_copy(g_hbm.at[sidx.at[b]], rows_v.at[b], gsem[b])
    pltpu.sync_copy(edges_hbm.at[1, wid], didx)
    r0 = s * RPT
    pltpu.sync_copy(g_hbm.at[pl.ds(r0, RPT)], acc.at[pl.ds(r0, RPT)])
    plsc.subcore_barrier()

    def fire_gather(b, k):
        pltpu.async_copy(g_hbm.at[sidx.at[b]], rows_v.at[k], gsem[k])

    def wait_gather(b, k):
        pltpu.make_async_copy(g_hbm.at[sidx.at[b]], rows_v.at[k], gsem[k]).wait()

    def fire_scatter(b, k):
        pltpu.async_copy(rows_v.at[k], acc.at[didx.at[b]], ssem[k], add=True)

    def wait_scatter(b, k):
        pltpu.make_async_copy(rows_v.at[k], acc.at[didx.at[b]], ssem[k]).wait()

    def step(b, k, do_ss_wait, do_g_fire):
        k2 = (k + GA) % NS
        if do_ss_wait:
            wait_scatter(b - GA, k2)
        if do_g_fire:
            fire_gather(b + GA, k2)
        wait_gather(b, k)
        fire_scatter(b, k)

    for b in range(GA):
        step(b, b, False, True)

    def octet(it, carry):
        b = GA + NS * it
        for j in range(NS):
            step(b + j, (GA + j) % NS, True, True)
        return carry

    lax.fori_loop(0, (NB - 2 * GA) // NS, octet, 0)
    for b in range(NB - GA, NB):
        step(b, b % NS, True, False)
    for b in range(NB - GA, NB):
        wait_scatter(b, b % NS)
    plsc.subcore_barrier()
    pltpu.sync_copy(acc.at[pl.ds(r0, RPT)], out_hbm.at[c, pl.ds(r0, RPT)])


_sc_prop = functools.partial(
    pl.kernel,
    out_type=jax.ShapeDtypeStruct((2, NPAD, DH), jnp.float32),
    mesh=_MESH,
    compiler_params=_SC_PARAMS,
    scratch_types=[
        pltpu.VMEM((NB, BB), jnp.int32),
        pltpu.VMEM((NB, BB), jnp.int32),
        pltpu.VMEM((8, BB, DH), jnp.float32),
        pltpu.VMEM_SHARED((NPAD, DH), jnp.float32),
    ] + [pltpu.SemaphoreType.DMA] * 16,
)(_prop_body)


def _tc_dis_body(d_ref, o_ref):
    o_ref[...] = lax.rsqrt(1.0 + d_ref[0] + d_ref[1])


def _tc_first_body(x_ref, w_ref, dis_ref, g_ref):
    g_ref[...] = dis_ref[...] * jnp.dot(
        x_ref[...], w_ref[...], preferred_element_type=jnp.float32
    )


def _tc_comb_body(a_ref, g_ref, dis_ref, b_ref, w_ref, o_ref):
    dis = dis_ref[...]
    t = jnp.maximum(dis * (a_ref[0] + a_ref[1] - g_ref[...]) + b_ref[...], 0.0)
    o_ref[...] = dis * jnp.dot(t, w_ref[...], preferred_element_type=jnp.float32)


def _tc_last_body(a_ref, g_ref, dis_ref, b_ref, o_ref):
    o_ref[...] = jnp.maximum(
        dis_ref[...] * (a_ref[0] + a_ref[1] - g_ref[...]) + b_ref[...], 0.0
    )


def _row_spec(cols=128):
    return pl.BlockSpec((RP, cols), lambda i: (i, 0))


def _pair_spec():
    return pl.BlockSpec((2, RP, 128), lambda i: (0, i, 0))


def _full_spec(r, c):
    return pl.BlockSpec((r, c), lambda i: (0, 0))


_PAIR_OUT = jax.ShapeDtypeStruct((NP2, 128), jnp.float32)

_tc_dis = pl.pallas_call(
    _tc_dis_body,
    grid=(NRB,),
    in_specs=[_pair_spec()],
    out_specs=_row_spec(),
    out_shape=_PAIR_OUT,
)

_tc_first = pl.pallas_call(
    _tc_first_body,
    grid=(NRB,),
    in_specs=[_row_spec(256), _full_spec(256, 128), _row_spec()],
    out_specs=_row_spec(),
    out_shape=_PAIR_OUT,
)

_tc_comb = pl.pallas_call(
    _tc_comb_body,
    grid=(NRB,),
    in_specs=[
        _pair_spec(), _row_spec(), _row_spec(),
        _full_spec(1, 128), _full_spec(128, 128),
    ],
    out_specs=_row_spec(),
    out_shape=_PAIR_OUT,
)

_tc_last = pl.pallas_call(
    _tc_last_body,
    grid=(NRB,),
    in_specs=[_pair_spec(), _row_spec(), _row_spec(), _full_spec(1, 128)],
    out_specs=_row_spec(),
    out_shape=_PAIR_OUT,
)


def _blockdiag(W):
    di, do = W.shape
    Wb = jnp.zeros((2 * di, 128), W.dtype)
    return Wb.at[:di, :do].set(W).at[di:, do:].set(W)


def kernel(x, edge_index, W0, b0, W1, b1, W2, b2, W3, b3, W4, b4):
    n_pad_rows = NPAD - N
    pad_ids = N + (jnp.arange(EPAD - E, dtype=jnp.int32) % n_pad_rows)
    edges4 = jnp.concatenate(
        [edge_index.astype(jnp.int32),
         jnp.broadcast_to(pad_ids, (2, EPAD - E))], axis=1
    ).reshape(2, NW, NB, BB)
    x_pair = jnp.pad(x, ((0, n_pad_rows), (0, 0))).reshape(NP2, 2 * DIN)
    ones = jnp.ones((BB, DH), jnp.float32)
    zeros = jnp.zeros((RPT, DH), jnp.float32)

    degp = _sc_deg(edges4, ones, zeros)
    dis = _tc_dis(degp.reshape(2, NP2, 128))

    g = _tc_first(x_pair, _blockdiag(W0), dis)
    Ws = [W1, W2, W3, W4]
    bs = [b0, b1, b2, b3, b4]
    for i in range(4):
        acc = _sc_prop(g.reshape(NPAD, DH), edges4)
        g = _tc_comb(acc.reshape(2, NP2, 128), g, dis,
                     jnp.concatenate([bs[i], bs[i]]).reshape(1, 128),
                     _blockdiag(Ws[i]))
    acc = _sc_prop(g.reshape(NPAD, DH), edges4)
    out = _tc_last(acc.reshape(2, NP2, 128), g, dis,
                   jnp.concatenate([bs[4], bs[4]]).reshape(1, 128))
    return out.reshape(NPAD, DH)[:N]
